# baseline (speedup 1.0000x reference)
import jax
import jax.numpy as jnp
from jax import lax
from jax.experimental import pallas as pl
from jax.experimental.pallas import tpu as pltpu

_BATCH = 16384
_INPUT_DIM = 1000
_EMBED_DIM = 16
_BN = 4096


def _body(xt_ref, e_ref, o_ref):
    o_ref[...] = lax.dot_general(
        e_ref[...], xt_ref[...],
        dimension_numbers=(((0,), (0,)), ((), ())),
        preferred_element_type=jnp.float32)


def kernel(x, embedding):
    xt = x.T
    grid = (_BATCH // _BN,)
    out_t = pl.pallas_call(
        _body,
        grid=grid,
        in_specs=[
            pl.BlockSpec((_INPUT_DIM, _BN), lambda i: (0, i)),
            pl.BlockSpec((_INPUT_DIM, _EMBED_DIM), lambda i: (0, 0)),
        ],
        out_specs=pl.BlockSpec((_EMBED_DIM, _BN), lambda i: (0, i)),
        out_shape=jax.ShapeDtypeStruct((_EMBED_DIM, _BATCH), jnp.float32),
        compiler_params=pltpu.CompilerParams(
            dimension_semantics=("parallel",),
        ),
    )(xt, embedding)
    return out_t.T
